# baseline (speedup 1.0000x reference)
import jax
import jax.numpy as jnp
from jax import lax
from jax.experimental import pallas as pl
from jax.experimental.pallas import tpu as pltpu
from jax.experimental.pallas import tpu_sc as plsc

NC, NS = 2, 16
NW = NC * NS

NX, DX = 100000, 32
NE, DE = 3200000, 16
VE = 1000

CX = 1024
NSUB = CX // 128
NCHX = (NX + CX - 1) // CX
XBLK = (NX + 127) // 128
XT = 8 * 8 * 128

NBLK = NE // 128
WBLK = 800
CE = 2048
NCHE = WBLK // 16
NG = CE // 16
EB_T = 16 * 8 * 128
ST = DE + 1


def _body(x_hbm, e_hbm, wx_hbm, we_hbm, outx_hbm, oute_hbm,
          xidx_v, xrows_v, etab_v, eidx_v, erows_v, sem, si0, si1, sw0, sw1):
    wid = lax.axis_index("s") * NC + lax.axis_index("c")

    lane = lax.iota(jnp.int32, 16)
    zero16 = jnp.zeros((16,), jnp.int32)
    xtile = erows_v.at[1]

    def xchunk(c, is_last):
        if is_last:
            pltpu.sync_copy(x_hbm.at[pl.ds(NX - 672, 640)],
                            xidx_v.at[pl.ds(0, 640)])
            pltpu.sync_copy(x_hbm.at[pl.ds(NX - 32, 32)],
                            xidx_v.at[pl.ds(640, 32)])
            for off in range(672, CX, 16):
                xidx_v[pl.ds(off, 16)] = zero16
        else:
            pltpu.sync_copy(x_hbm.at[pl.ds(c * CX, CX)], xidx_v)
        cps = [pltpu.async_copy(wx_hbm.at[xidx_v.at[pl.ds(j * 128, 128)]],
                                xrows_v.at[pl.ds(j * 128, 128)], sem)
               for j in range(NSUB)]
        for cp in cps:
            cp.wait()

        @plsc.parallel_loop(0, CX // 16, unroll=2)
        def _(g):
            rows16 = lane + g * 16
            dstg = (g // 8) * 1024 + (g % 8) * 16
            for t in range(4):
                for s in range(8):
                    vals = plsc.load_gather(
                        xrows_v, [rows16, jnp.full((16,), 8 * t + s, jnp.int32)])
                    xtile[pl.ds(dstg + t * XT + s * 128, 16)] = vals

        nb = 6 if is_last else 8
        wcps = [pltpu.async_copy(
            xtile.at[pl.ds(t * XT, nb * 1024)],
            outx_hbm.at[pl.ds((t * XBLK + c * 8) * 1024, nb * 1024)], sem)
            for t in range(4)]
        for cp in wcps:
            cp.wait()

    for i in range(4):
        c = wid + NW * i

        @pl.when(c < NCHX - 1)
        def _():
            xchunk(c, False)

        @pl.when(c == NCHX - 1)
        def _():
            xchunk(NCHX - 1, True)

    pltpu.sync_copy(we_hbm, erows_v.at[0, pl.ds(0, VE * DE)])

    def trow(r, carry):
        etab_v[pl.ds(r * ST, DE)] = erows_v[0, pl.ds(r * DE, DE)]
        return carry

    lax.fori_loop(0, VE, trow, 0)
    wblk0 = jnp.minimum(wid * WBLK, NBLK - WBLK)
    si = (si0, si1)
    sw = (sw0, sw1)

    def idx_start(i, bb):
        b = pl.multiple_of((wblk0 + i * 16) * 128, 8)
        pltpu.async_copy(e_hbm.at[pl.ds(b, CE)], eidx_v.at[bb], si[bb])

    def idx_wait(bb):
        pltpu.make_async_copy(e_hbm.at[pl.ds(0, CE)], eidx_v.at[bb],
                              si[bb]).wait()

    def write_start(i, bb):
        blk = wblk0 + i * 16
        for t in range(2):
            pltpu.async_copy(
                erows_v.at[bb, pl.ds(t * EB_T, EB_T)],
                oute_hbm.at[pl.ds((t * NBLK + blk) * 1024, EB_T)], sw[bb])

    def write_wait(bb):
        pltpu.make_async_copy(oute_hbm.at[pl.ds(0, 2 * EB_T)],
                              erows_v.at[bb], sw[bb]).wait()

    def compute(bb):
        @plsc.parallel_loop(0, NG, unroll=4)
        def _(g):
            rows16 = eidx_v[bb, pl.ds(g * 16, 16)]
            ridx = rows16 * ST
            ridx_p = [ridx + c if c else ridx for c in range(8)]
            dst = (g // 8) * 1024 + (g % 8) * 16
            for j in range(DE):
                vals = plsc.load_gather(
                    etab_v.at[pl.ds(8 * (j // 8), VE * ST - 8)],
                    [ridx_p[j % 8]])
                erows_v[bb, pl.ds(dst + (j // 8) * EB_T + (j % 8) * 128, 16)] = vals

    idx_start(0, 0)

    def pipe(it, carry):
        for b in range(2):
            i = 2 * it + b

            @pl.when(i + 1 < NCHE)
            def _():
                idx_start(i + 1, 1 - b)

            idx_wait(b)

            @pl.when(i >= 2)
            def _():
                write_wait(b)

            compute(b)
            write_start(i, b)
        return carry

    lax.fori_loop(0, NCHE // 2, pipe, 0)
    write_wait(0)
    write_wait(1)


def kernel(x, edge_attr, embed_x_W, embed_edge_W):
    mesh = plsc.VectorSubcoreMesh(core_axis_name="c", subcore_axis_name="s")
    f = pl.kernel(
        _body,
        out_type=[jax.ShapeDtypeStruct((4 * XBLK * 1024,), jnp.float32),
                  jax.ShapeDtypeStruct((NE * DE,), jnp.float32)],
        mesh=mesh,
        compiler_params=pltpu.CompilerParams(
            use_tc_tiling_on_sc=False, needs_layout_passes=False),
        scratch_types=[
            pltpu.VMEM((CX,), jnp.int32),
            pltpu.VMEM((CX, DX), jnp.float32),
            pltpu.VMEM((VE * ST,), jnp.float32),
            pltpu.VMEM((2, CE), jnp.int32),
            pltpu.VMEM((2, 2 * EB_T), jnp.float32),
            pltpu.SemaphoreType.DMA,
            pltpu.SemaphoreType.DMA,
            pltpu.SemaphoreType.DMA,
            pltpu.SemaphoreType.DMA,
            pltpu.SemaphoreType.DMA,
        ],
    )
    out_x, out_e = f(x, edge_attr, embed_x_W, embed_edge_W.reshape(-1))
    out_x = out_x.reshape(4, XBLK, 8, 128).transpose(1, 3, 0, 2)
    out_x = out_x.reshape(XBLK * 128, DX)[:NX]
    out_e = out_e.reshape(2, NE // 128, 8, 128).transpose(1, 3, 0, 2)
    return (out_x, out_e.reshape(NE, DE))


# speedup vs baseline: 1.1981x; 1.1981x over previous
import jax
import jax.numpy as jnp
from jax import lax
from jax.experimental import pallas as pl
from jax.experimental.pallas import tpu as pltpu
from jax.experimental.pallas import tpu_sc as plsc

NC, NS = 2, 16
NW = NC * NS

NX, DX = 100000, 32
NE, DE = 3200000, 16
VE = 1000

CX = 512
NSUB = CX // 128
NCHX = (NX + CX - 1) // CX
XBLK = (NX + 127) // 128
XT = 4 * 8 * 128
XVALID = NX - (NCHX - 1) * CX

NBLK = NE // 128
WBLK = 800
CE = 2048
NCHE = WBLK // 16
NG = CE // 16
EB_T = 16 * 8 * 128
ST = DE + 1


def _body(x_hbm, e_hbm, wx_hbm, we_hbm, outx_hbm, oute_hbm,
          xidx_v, xrows_v, xpad_v, etab_v, eidx_v, erows_v,
          sem, si0, si1, sw0, sw1):
    wid = lax.axis_index("s") * NC + lax.axis_index("c")

    lane = lax.iota(jnp.int32, 16)
    zero16 = jnp.zeros((16,), jnp.int32)
    xtile = erows_v.at[1]

    def xchunk(c, is_last):
        if is_last:
            pltpu.sync_copy(x_hbm.at[pl.ds(NX - XVALID, 128)],
                            xidx_v.at[pl.ds(0, 128)])
            pltpu.sync_copy(x_hbm.at[pl.ds(NX - 32, 32)],
                            xidx_v.at[pl.ds(128, 32)])
            for off in range(XVALID, CX, 16):
                xidx_v[pl.ds(off, 16)] = zero16
        else:
            pltpu.sync_copy(x_hbm.at[pl.ds(c * CX, CX)], xidx_v)
        cps = [pltpu.async_copy(wx_hbm.at[xidx_v.at[pl.ds(j * 128, 128)]],
                                xrows_v.at[pl.ds(j * 128, 128)], sem)
               for j in range(NSUB)]
        for cp in cps:
            cp.wait()

        @plsc.parallel_loop(0, CX, unroll=4)
        def _(r):
            xpad_v[r, pl.ds(0, 16)] = xrows_v[r, pl.ds(0, 16)]
            xpad_v[r, pl.ds(16, 16)] = xrows_v[r, pl.ds(16, 16)]

        @plsc.parallel_loop(0, CX // 16, unroll=2)
        def _(g):
            rows16 = lane + g * 16
            dstg = (g // 8) * 1024 + (g % 8) * 16
            for t in range(4):
                for s in range(8):
                    vals = plsc.load_gather(
                        xpad_v, [rows16, jnp.full((16,), 8 * t + s, jnp.int32)])
                    xtile[pl.ds(dstg + t * XT + s * 128, 16)] = vals

        nb = 2 if is_last else 4
        wcps = [pltpu.async_copy(
            xtile.at[pl.ds(t * XT, nb * 1024)],
            outx_hbm.at[pl.ds((t * XBLK + c * 4) * 1024, nb * 1024)], sem)
            for t in range(4)]
        for cp in wcps:
            cp.wait()

    for i in range(7):
        c = wid + NW * i

        @pl.when(c < NCHX - 1)
        def _():
            xchunk(c, False)

        @pl.when(c == NCHX - 1)
        def _():
            xchunk(NCHX - 1, True)

    pltpu.sync_copy(we_hbm, erows_v.at[0, pl.ds(0, VE * DE)])

    def trow(r, carry):
        etab_v[pl.ds(r * ST, DE)] = erows_v[0, pl.ds(r * DE, DE)]
        return carry

    lax.fori_loop(0, VE, trow, 0)
    wblk0 = jnp.minimum(wid * WBLK, NBLK - WBLK)
    si = (si0, si1)
    sw = (sw0, sw1)

    def idx_start(i, bb):
        b = pl.multiple_of((wblk0 + i * 16) * 128, 8)
        pltpu.async_copy(e_hbm.at[pl.ds(b, CE)], eidx_v.at[bb], si[bb])

    def idx_wait(bb):
        pltpu.make_async_copy(e_hbm.at[pl.ds(0, CE)], eidx_v.at[bb],
                              si[bb]).wait()

    def write_start(i, bb):
        blk = wblk0 + i * 16
        for t in range(2):
            pltpu.async_copy(
                erows_v.at[bb, pl.ds(t * EB_T, EB_T)],
                oute_hbm.at[pl.ds((t * NBLK + blk) * 1024, EB_T)], sw[bb])

    def write_wait(bb):
        pltpu.make_async_copy(oute_hbm.at[pl.ds(0, 2 * EB_T)],
                              erows_v.at[bb], sw[bb]).wait()

    def compute(bb):
        @plsc.parallel_loop(0, NG, unroll=4)
        def _(g):
            rows16 = eidx_v[bb, pl.ds(g * 16, 16)]
            ridx = rows16 * ST
            ridx_p = [ridx + c if c else ridx for c in range(8)]
            dst = (g // 8) * 1024 + (g % 8) * 16
            for j in range(DE):
                vals = plsc.load_gather(
                    etab_v.at[pl.ds(8 * (j // 8), VE * ST - 8)],
                    [ridx_p[j % 8]])
                erows_v[bb, pl.ds(dst + (j // 8) * EB_T + (j % 8) * 128, 16)] = vals

    idx_start(0, 0)

    def pipe(it, carry):
        for b in range(2):
            i = 2 * it + b

            @pl.when(i + 1 < NCHE)
            def _():
                idx_start(i + 1, 1 - b)

            idx_wait(b)

            @pl.when(i >= 2)
            def _():
                write_wait(b)

            compute(b)
            write_start(i, b)
        return carry

    lax.fori_loop(0, NCHE // 2, pipe, 0)
    write_wait(0)
    write_wait(1)


def kernel(x, edge_attr, embed_x_W, embed_edge_W):
    mesh = plsc.VectorSubcoreMesh(core_axis_name="c", subcore_axis_name="s")
    f = pl.kernel(
        _body,
        out_type=[jax.ShapeDtypeStruct((4 * XBLK * 1024,), jnp.float32),
                  jax.ShapeDtypeStruct((NE * DE,), jnp.float32)],
        mesh=mesh,
        compiler_params=pltpu.CompilerParams(
            use_tc_tiling_on_sc=False, needs_layout_passes=False),
        scratch_types=[
            pltpu.VMEM((CX,), jnp.int32),
            pltpu.VMEM((CX, DX), jnp.float32),
            pltpu.VMEM((CX, DX + 1), jnp.float32),
            pltpu.VMEM((VE * ST,), jnp.float32),
            pltpu.VMEM((2, CE), jnp.int32),
            pltpu.VMEM((2, 2 * EB_T), jnp.float32),
            pltpu.SemaphoreType.DMA,
            pltpu.SemaphoreType.DMA,
            pltpu.SemaphoreType.DMA,
            pltpu.SemaphoreType.DMA,
            pltpu.SemaphoreType.DMA,
        ],
    )
    out_x, out_e = f(x, edge_attr, embed_x_W, embed_edge_W.reshape(-1))
    out_x = out_x.reshape(4, XBLK, 8, 128).transpose(1, 3, 0, 2)
    out_x = out_x.reshape(XBLK * 128, DX)[:NX]
    out_e = out_e.reshape(2, NE // 128, 8, 128).transpose(1, 3, 0, 2)
    return (out_x, out_e.reshape(NE, DE))


# speedup vs baseline: 1.3005x; 1.0855x over previous
import jax
import jax.numpy as jnp
from jax import lax
from jax.experimental import pallas as pl
from jax.experimental.pallas import tpu as pltpu
from jax.experimental.pallas import tpu_sc as plsc

NC, NS = 2, 16
NW = NC * NS

NX, DX = 100000, 32
NE, DE = 3200000, 16
VE = 1000

CX = 512
NSUB = CX // 128
NCHX = (NX + CX - 1) // CX
XBLK = (NX + 127) // 128
XT = 4 * 8 * 128
XVALID = NX - (NCHX - 1) * CX

NBLK = NE // 128
WBLK = 800
CE = 2048
NCHE = WBLK // 16
NG = CE // 16
EB_T = 16 * 8 * 128
ST = DE + 1


def _edge_body(e_hbm, we_hbm, oute_hbm, etab_v, eidx_v, erows_v,
               si0, si1, sw0, sw1):
    wid = lax.axis_index("s") * NC + lax.axis_index("c")
    pltpu.sync_copy(we_hbm, erows_v.at[0, pl.ds(0, VE * DE)])

    def trow(r, carry):
        etab_v[pl.ds(r * ST, DE)] = erows_v[0, pl.ds(r * DE, DE)]
        return carry

    lax.fori_loop(0, VE, trow, 0)
    wblk0 = jnp.minimum(wid * WBLK, NBLK - WBLK)
    si = (si0, si1)
    sw = (sw0, sw1)

    def idx_start(i, bb):
        b = pl.multiple_of((wblk0 + i * 16) * 128, 8)
        pltpu.async_copy(e_hbm.at[pl.ds(b, CE)], eidx_v.at[bb], si[bb])

    def idx_wait(bb):
        pltpu.make_async_copy(e_hbm.at[pl.ds(0, CE)], eidx_v.at[bb],
                              si[bb]).wait()

    def write_start(i, bb):
        blk = wblk0 + i * 16
        for t in range(2):
            pltpu.async_copy(
                erows_v.at[bb, pl.ds(t * EB_T, EB_T)],
                oute_hbm.at[pl.ds((t * NBLK + blk) * 1024, EB_T)], sw[bb])

    def write_wait(bb):
        pltpu.make_async_copy(oute_hbm.at[pl.ds(0, 2 * EB_T)],
                              erows_v.at[bb], sw[bb]).wait()

    def compute(bb):
        @plsc.parallel_loop(0, NG, unroll=4)
        def _(g):
            rows16 = eidx_v[bb, pl.ds(g * 16, 16)]
            ridx = rows16 * ST
            ridx_p = [ridx + c if c else ridx for c in range(8)]
            dst = (g // 8) * 1024 + (g % 8) * 16
            for j in range(DE):
                vals = plsc.load_gather(
                    etab_v.at[pl.ds(8 * (j // 8), VE * ST - 8)],
                    [ridx_p[j % 8]])
                erows_v[bb, pl.ds(dst + (j // 8) * EB_T + (j % 8) * 128, 16)] = vals

    idx_start(0, 0)

    def pipe(it, carry):
        for b in range(2):
            i = 2 * it + b

            @pl.when(i + 1 < NCHE)
            def _():
                idx_start(i + 1, 1 - b)

            idx_wait(b)

            @pl.when(i >= 2)
            def _():
                write_wait(b)

            compute(b)
            write_start(i, b)
        return carry

    lax.fori_loop(0, NCHE // 2, pipe, 0)
    write_wait(0)
    write_wait(1)


def _x_body(x_hbm, wx_hbm, dep_hbm, outx_hbm,
            xidx_v, xrows_v, xpad_v, xtile_v, sem):
    wid = lax.axis_index("s") * NC + lax.axis_index("c")
    lane = lax.iota(jnp.int32, 16)
    zero16 = jnp.zeros((16,), jnp.int32)

    def xchunk(c, is_last):
        if is_last:
            pltpu.sync_copy(x_hbm.at[pl.ds(NX - XVALID, 128)],
                            xidx_v.at[pl.ds(0, 128)])
            pltpu.sync_copy(x_hbm.at[pl.ds(NX - 32, 32)],
                            xidx_v.at[pl.ds(128, 32)])
            for off in range(XVALID, CX, 16):
                xidx_v[pl.ds(off, 16)] = zero16
        else:
            pltpu.sync_copy(x_hbm.at[pl.ds(c * CX, CX)], xidx_v)
        cps = [pltpu.async_copy(wx_hbm.at[xidx_v.at[pl.ds(j * 128, 128)]],
                                xrows_v.at[pl.ds(j * 128, 128)], sem)
               for j in range(NSUB)]
        for cp in cps:
            cp.wait()

        @plsc.parallel_loop(0, CX, unroll=4)
        def _(r):
            xpad_v[r, pl.ds(0, 16)] = xrows_v[r, pl.ds(0, 16)]
            xpad_v[r, pl.ds(16, 16)] = xrows_v[r, pl.ds(16, 16)]

        @plsc.parallel_loop(0, CX // 16, unroll=2)
        def _(g):
            rows16 = lane + g * 16
            dstg = (g // 8) * 1024 + (g % 8) * 16
            for t in range(4):
                for s in range(8):
                    vals = plsc.load_gather(
                        xpad_v, [rows16, jnp.full((16,), 8 * t + s, jnp.int32)])
                    xtile_v[pl.ds(dstg + t * XT + s * 128, 16)] = vals

        nb = 2 if is_last else 4
        wcps = [pltpu.async_copy(
            xtile_v.at[pl.ds(t * XT, nb * 1024)],
            outx_hbm.at[pl.ds((t * XBLK + c * 4) * 1024, nb * 1024)], sem)
            for t in range(4)]
        for cp in wcps:
            cp.wait()

    for i in range(7):
        c = wid + NW * i

        @pl.when(c < NCHX - 1)
        def _():
            xchunk(c, False)

        @pl.when(c == NCHX - 1)
        def _():
            xchunk(NCHX - 1, True)


def kernel(x, edge_attr, embed_x_W, embed_edge_W):
    mesh = plsc.VectorSubcoreMesh(core_axis_name="c", subcore_axis_name="s")
    params = pltpu.CompilerParams(
        use_tc_tiling_on_sc=False, needs_layout_passes=False)

    f_edge = pl.kernel(
        _edge_body,
        out_type=jax.ShapeDtypeStruct((NE * DE,), jnp.float32),
        mesh=mesh,
        compiler_params=params,
        scratch_types=[
            pltpu.VMEM((VE * ST,), jnp.float32),
            pltpu.VMEM((2, CE), jnp.int32),
            pltpu.VMEM((2, 2 * EB_T), jnp.float32),
            pltpu.SemaphoreType.DMA,
            pltpu.SemaphoreType.DMA,
            pltpu.SemaphoreType.DMA,
            pltpu.SemaphoreType.DMA,
        ],
    )
    f_x = pl.kernel(
        _x_body,
        out_type=jax.ShapeDtypeStruct((4 * XBLK * 1024,), jnp.float32),
        mesh=mesh,
        compiler_params=params,
        scratch_types=[
            pltpu.VMEM((CX,), jnp.int32),
            pltpu.VMEM((CX, DX), jnp.float32),
            pltpu.VMEM((CX, DX + 1), jnp.float32),
            pltpu.VMEM((4 * XT,), jnp.float32),
            pltpu.SemaphoreType.DMA,
        ],
    )

    out_e = f_edge(edge_attr, embed_edge_W.reshape(-1))
    out_x = f_x(x, embed_x_W, out_e[:8])
    out_x = out_x.reshape(4, XBLK, 8, 128).transpose(1, 3, 0, 2)
    out_x = out_x.reshape(XBLK * 128, DX)[:NX]
    out_e = out_e.reshape(2, NE // 128, 8, 128).transpose(1, 3, 0, 2)
    return (out_x, out_e.reshape(NE, DE))


# speedup vs baseline: 1.3678x; 1.0518x over previous
import jax
import jax.numpy as jnp
from jax import lax
from jax.experimental import pallas as pl
from jax.experimental.pallas import tpu as pltpu
from jax.experimental.pallas import tpu_sc as plsc

NC, NS = 2, 16
NW = NC * NS

NX, DX = 100000, 32
NE, DE = 3200000, 16
VE = 1000

CX = 512
NSUB = CX // 128
NCHX = (NX + CX - 1) // CX
XBLK = (NX + 127) // 128
XT = 4 * 8 * 128
XVALID = NX - (NCHX - 1) * CX

NBLK = NE // 128
WBLK = 800
CE = 2048
NCHE = WBLK // 16
NG = CE // 16
EB_T = 16 * 8 * 128
ST = DE + 1


def _edge_body(e_hbm, we_hbm, oute_hbm, etab_v, eidx_v, erows_v,
               si0, si1, sw0, sw1):
    wid = lax.axis_index("s") * NC + lax.axis_index("c")
    pltpu.sync_copy(we_hbm, erows_v.at[0, pl.ds(0, VE * DE)])

    def trow(r, carry):
        etab_v[pl.ds(r * ST, DE)] = erows_v[0, pl.ds(r * DE, DE)]
        return carry

    lax.fori_loop(0, VE, trow, 0)
    wblk0 = jnp.minimum(wid * WBLK, NBLK - WBLK)
    si = (si0, si1)
    sw = (sw0, sw1)

    def idx_start(i, bb):
        b = pl.multiple_of((wblk0 + i * 16) * 128, 8)
        pltpu.async_copy(e_hbm.at[pl.ds(b, CE)], eidx_v.at[bb], si[bb])

    def idx_wait(bb):
        pltpu.make_async_copy(e_hbm.at[pl.ds(0, CE)], eidx_v.at[bb],
                              si[bb]).wait()

    def write_start(i, bb):
        blk = wblk0 + i * 16
        for t in range(2):
            pltpu.async_copy(
                erows_v.at[bb, pl.ds(t * EB_T, EB_T)],
                oute_hbm.at[pl.ds((t * NBLK + blk) * 1024, EB_T)], sw[bb])

    def write_wait(bb):
        pltpu.make_async_copy(oute_hbm.at[pl.ds(0, 2 * EB_T)],
                              erows_v.at[bb], sw[bb]).wait()

    def compute(bb):
        @plsc.parallel_loop(0, NG, unroll=4)
        def _(g):
            rows16 = eidx_v[bb, pl.ds(g * 16, 16)]
            ridx = rows16 * ST
            ridx_p = [ridx + c if c else ridx for c in range(8)]
            dst = (g // 8) * 1024 + (g % 8) * 16
            for j in range(DE):
                vals = plsc.load_gather(
                    etab_v.at[pl.ds(8 * (j // 8), VE * ST - 8)],
                    [ridx_p[j % 8]])
                erows_v[bb, pl.ds(dst + (j // 8) * EB_T + (j % 8) * 128, 16)] = vals

    idx_start(0, 0)

    def pipe(it, carry):
        for b in range(2):
            i = 2 * it + b

            @pl.when(i + 1 < NCHE)
            def _():
                idx_start(i + 1, 1 - b)

            idx_wait(b)

            @pl.when(i >= 2)
            def _():
                write_wait(b)

            compute(b)
            write_start(i, b)
        return carry

    lax.fori_loop(0, NCHE // 2, pipe, 0)
    write_wait(0)
    write_wait(1)


def _x_body(x_hbm, wx_hbm, dep_hbm, outx_hbm,
            xidx_v, xrows_v, xpad_v, xtile_v, sg0, sg1, sv0, sv1):
    wid = lax.axis_index("s") * NC + lax.axis_index("c")
    lane = lax.iota(jnp.int32, 16)
    zero16 = jnp.zeros((16,), jnp.int32)
    sg = (sg0, sg1)
    sv = (sv0, sv1)

    def fire(c, b):
        @pl.when(c < NCHX - 1)
        def _():
            pltpu.sync_copy(x_hbm.at[pl.ds(c * CX, CX)], xidx_v.at[b])
            for j in range(NSUB):
                pltpu.async_copy(
                    wx_hbm.at[xidx_v.at[b, pl.ds(j * 128, 128)]],
                    xrows_v.at[b, pl.ds(j * 128, 128)], sg[b])

    def transpose(b):
        @plsc.parallel_loop(0, CX, unroll=4)
        def _(r):
            xpad_v[r, pl.ds(0, 16)] = xrows_v[b, r, pl.ds(0, 16)]
            xpad_v[r, pl.ds(16, 16)] = xrows_v[b, r, pl.ds(16, 16)]

        @plsc.parallel_loop(0, CX // 16, unroll=2)
        def _(g):
            rows16 = lane + g * 16
            dstg = (g // 8) * 1024 + (g % 8) * 16
            for t in range(4):
                for s in range(8):
                    vals = plsc.load_gather(
                        xpad_v, [rows16, jnp.full((16,), 8 * t + s, jnp.int32)])
                    xtile_v[b, pl.ds(dstg + t * XT + s * 128, 16)] = vals

    def process(c, b, drain_prev):
        def go():
            pltpu.make_async_copy(wx_hbm.at[pl.ds(0, CX)], xrows_v.at[b],
                                  sg[b]).wait()
            transpose(b)
            if drain_prev:
                pltpu.make_async_copy(outx_hbm.at[pl.ds(0, 4 * XT)],
                                      xtile_v.at[b], sv[b]).wait()
            for t in range(4):
                pltpu.async_copy(
                    xtile_v.at[b, pl.ds(t * XT, 4 * 1024)],
                    outx_hbm.at[pl.ds((t * XBLK + c * 4) * 1024, 4 * 1024)],
                    sv[b])
        return go

    fire(wid, 0)
    for i in range(7):
        c = wid + NW * i
        fire(c + NW, 1 - i % 2)
        go = process(c, i % 2, i >= 2)
        if i < 6:
            go()
        else:
            pl.when(c < NCHX - 1)(go)
    for b in range(2):
        pltpu.make_async_copy(outx_hbm.at[pl.ds(0, 4 * XT)],
                              xtile_v.at[b], sv[b]).wait()

    @pl.when(wid == (NCHX - 1) % NW)
    def _():
        b = 0
        pltpu.sync_copy(x_hbm.at[pl.ds(NX - XVALID, 128)],
                        xidx_v.at[b, pl.ds(0, 128)])
        pltpu.sync_copy(x_hbm.at[pl.ds(NX - 32, 32)],
                        xidx_v.at[b, pl.ds(128, 32)])
        for off in range(XVALID, CX, 16):
            xidx_v[b, pl.ds(off, 16)] = zero16
        for j in range(NSUB):
            pltpu.async_copy(wx_hbm.at[xidx_v.at[b, pl.ds(j * 128, 128)]],
                             xrows_v.at[b, pl.ds(j * 128, 128)], sg[b])
        pltpu.make_async_copy(wx_hbm.at[pl.ds(0, CX)], xrows_v.at[b],
                              sg[b]).wait()
        transpose(b)
        for t in range(4):
            pltpu.sync_copy(
                xtile_v.at[b, pl.ds(t * XT, 2 * 1024)],
                outx_hbm.at[pl.ds((t * XBLK + (NCHX - 1) * 4) * 1024,
                                  2 * 1024)])


def kernel(x, edge_attr, embed_x_W, embed_edge_W):
    mesh = plsc.VectorSubcoreMesh(core_axis_name="c", subcore_axis_name="s")
    params = pltpu.CompilerParams(
        use_tc_tiling_on_sc=False, needs_layout_passes=False)

    f_edge = pl.kernel(
        _edge_body,
        out_type=jax.ShapeDtypeStruct((NE * DE,), jnp.float32),
        mesh=mesh,
        compiler_params=params,
        scratch_types=[
            pltpu.VMEM((VE * ST,), jnp.float32),
            pltpu.VMEM((2, CE), jnp.int32),
            pltpu.VMEM((2, 2 * EB_T), jnp.float32),
            pltpu.SemaphoreType.DMA,
            pltpu.SemaphoreType.DMA,
            pltpu.SemaphoreType.DMA,
            pltpu.SemaphoreType.DMA,
        ],
    )
    f_x = pl.kernel(
        _x_body,
        out_type=jax.ShapeDtypeStruct((4 * XBLK * 1024,), jnp.float32),
        mesh=mesh,
        compiler_params=params,
        scratch_types=[
            pltpu.VMEM((2, CX), jnp.int32),
            pltpu.VMEM((2, CX, DX), jnp.float32),
            pltpu.VMEM((CX, DX + 1), jnp.float32),
            pltpu.VMEM((2, 4 * XT), jnp.float32),
            pltpu.SemaphoreType.DMA,
            pltpu.SemaphoreType.DMA,
            pltpu.SemaphoreType.DMA,
            pltpu.SemaphoreType.DMA,
        ],
    )

    out_e = f_edge(edge_attr, embed_edge_W.reshape(-1))
    out_x = f_x(x, embed_x_W, out_e[:8])
    out_x = out_x.reshape(4, XBLK, 8, 128).transpose(1, 3, 0, 2)
    out_x = out_x.reshape(XBLK * 128, DX)[:NX]
    out_e = out_e.reshape(2, NE // 128, 8, 128).transpose(1, 3, 0, 2)
    return (out_x, out_e.reshape(NE, DE))
